# Initial kernel scaffold; baseline (speedup 1.0000x reference)
import jax
import jax.numpy as jnp
from jax.experimental import pallas as pl


def kernel(adj_indices, adj_values, image_adj_indices, image_adj_values, text_adj_indices, text_adj_values, u_embs, i_embs, image_embedding, text_embedding, image_W, image_b, text_W, text_b, modal_weight):
    raise NotImplementedError("write your pallas kernel here")


# speedup vs baseline: 1.4724x; 1.4724x over previous
import functools

import jax
import jax.numpy as jnp
from jax.experimental import pallas as pl
from jax.experimental.pallas import tpu as pltpu

U = 5000
I = 5000
N = 10000
D = 256
IMG_DIM = 4096
TXT_DIM = 768
MODAL_ADJ_W = 0.2

_MB = 1000


def _dense_body(w_ref, img_ref, iw_ref, ib_ref, txt_ref, tw_ref, tb_ref, out_ref):
    w0 = w_ref[0, 0]
    w1 = w_ref[0, 1]
    imf = jnp.dot(img_ref[...], iw_ref[...],
                  preferred_element_type=jnp.float32) + ib_ref[...]
    inorm = jnp.maximum(jnp.sqrt(jnp.sum(imf * imf, axis=1, keepdims=True)), 1e-12)
    txf = jnp.dot(txt_ref[...], tw_ref[...],
                  preferred_element_type=jnp.float32) + tb_ref[...]
    tnorm = jnp.maximum(jnp.sqrt(jnp.sum(txf * txf, axis=1, keepdims=True)), 1e-12)
    out_ref[...] = (w0 / inorm) * imf + (w1 / tnorm) * txf


def _modal_lower(weight, image_embedding, image_W, image_b, text_W, text_b,
                 text_embedding):
    grid = (I // _MB,)
    return pl.pallas_call(
        _dense_body,
        grid=grid,
        in_specs=[
            pl.BlockSpec(memory_space=pltpu.SMEM),
            pl.BlockSpec((_MB, IMG_DIM), lambda m: (m, 0)),
            pl.BlockSpec((IMG_DIM, D), lambda m: (0, 0)),
            pl.BlockSpec((1, D), lambda m: (0, 0)),
            pl.BlockSpec((_MB, TXT_DIM), lambda m: (m, 0)),
            pl.BlockSpec((TXT_DIM, D), lambda m: (0, 0)),
            pl.BlockSpec((1, D), lambda m: (0, 0)),
        ],
        out_specs=pl.BlockSpec((_MB, D), lambda m: (m, 0)),
        out_shape=jax.ShapeDtypeStruct((I, D), jnp.float32),
    )(weight.reshape(1, 2), image_embedding, image_W, image_b.reshape(1, D),
      text_embedding, text_W, text_b.reshape(1, D))


def _spmm(idx, vals, x, n):
    return jax.ops.segment_sum(vals[:, None] * x[idx[1]], idx[0], num_segments=n)


def kernel(adj_indices, adj_values, image_adj_indices, image_adj_values,
           text_adj_indices, text_adj_values, u_embs, i_embs, image_embedding,
           text_embedding, image_W, image_b, text_W, text_b, modal_weight):
    weight = jax.nn.softmax(modal_weight, axis=0)
    w0, w1 = weight[0], weight[1]
    ws = w0 + w1

    lower = _modal_lower(weight, image_embedding, image_W, image_b,
                         text_W, text_b, text_embedding)

    x1 = jnp.concatenate([ws * u_embs, lower], axis=0)
    base = jnp.concatenate([u_embs, i_embs], axis=0)

    z = _spmm(adj_indices, adj_values, x1, N)
    p = _spmm(image_adj_indices, image_adj_values, base, N)
    q = _spmm(text_adj_indices, text_adj_values, base, N)
    y2 = jnp.concatenate([z[:U], ws * i_embs], axis=0)
    h = _spmm(adj_indices, adj_values, y2, N)
    return z + h + MODAL_ADJ_W * (w0 * p + w1 * q)


# speedup vs baseline: 3.6342x; 2.4682x over previous
import functools

import jax
import jax.numpy as jnp
from jax import lax
from jax.experimental import pallas as pl
from jax.experimental.pallas import tpu as pltpu
from jax.experimental.pallas import tpu_sc as plsc

U = 5000
I = 5000
N = 10000
D = 256
HD = 128
IMG_DIM = 4096
TXT_DIM = 768
NNZ = 160000
MODAL_ADJ_W = 0.2

_MB = 1000

_BLK = 128
_PT_A = 80
_PT_B = 160
_PT_H = 80
_NT = 16
_EA = _PT_A * _NT * _BLK
_EB = _PT_B * _NT * _BLK
_RPT = 624


def _dense_body(w_ref, img_ref, iw_ref, ib_ref, txt_ref, tw_ref, tb_ref, out_ref):
    w0 = w_ref[0, 0]
    w1 = w_ref[0, 1]
    imf = jnp.dot(img_ref[...], iw_ref[...],
                  preferred_element_type=jnp.float32) + ib_ref[...]
    inorm = jnp.maximum(jnp.sqrt(jnp.sum(imf * imf, axis=1, keepdims=True)), 1e-12)
    txf = jnp.dot(txt_ref[...], tw_ref[...],
                  preferred_element_type=jnp.float32) + tb_ref[...]
    tnorm = jnp.maximum(jnp.sqrt(jnp.sum(txf * txf, axis=1, keepdims=True)), 1e-12)
    out_ref[...] = (w0 / inorm) * imf + (w1 / tnorm) * txf


def _modal_lower(weight, image_embedding, image_W, image_b, text_W, text_b,
                 text_embedding):
    return pl.pallas_call(
        _dense_body,
        grid=(I // _MB,),
        in_specs=[
            pl.BlockSpec(memory_space=pltpu.SMEM),
            pl.BlockSpec((_MB, IMG_DIM), lambda m: (m, 0)),
            pl.BlockSpec((IMG_DIM, D), lambda m: (0, 0)),
            pl.BlockSpec((1, D), lambda m: (0, 0)),
            pl.BlockSpec((_MB, TXT_DIM), lambda m: (m, 0)),
            pl.BlockSpec((TXT_DIM, D), lambda m: (0, 0)),
            pl.BlockSpec((1, D), lambda m: (0, 0)),
        ],
        out_specs=pl.BlockSpec((_MB, D), lambda m: (m, 0)),
        out_shape=jax.ShapeDtypeStruct((I, D), jnp.float32),
    )(weight.reshape(1, 2), image_embedding, image_W, image_b.reshape(1, D),
      text_embedding, text_W, text_b.reshape(1, D))


def _sc_body(tab1, i2s, sA, dA, vA, sB, dB, vB, sH,
             out, y2,
             srcb, dstb, valb, rows0, rows1, acc, g0, g1, l0, l1):
    cid = lax.axis_index("c")
    sid = lax.axis_index("s")
    rows = (rows0, rows1)
    gsem = (g0, g1)
    lsem = (l0, l1)

    z16 = jnp.zeros((16,), jnp.float32)

    def zrow(i, c):
        for j in range(8):
            rows0[i, pl.ds(j * 16, 16)] = z16
        return c

    lax.fori_loop(0, _BLK, zrow, 0)
    r0 = sid * _RPT
    for i in range(4):
        pltpu.sync_copy(rows0, acc.at[pl.ds(r0 + i * 128, 128)])
    pltpu.sync_copy(rows0.at[pl.ds(0, 112)], acc.at[pl.ds(r0 + 512, 112)])

    @pl.when(sid == 15)
    def _():
        pltpu.sync_copy(rows0.at[pl.ds(0, 16)], acc.at[pl.ds(N - 16, 16)])

    def fill_y2_lower(off, sz):
        so = cid * I + off
        do = cid * N + U + off
        pltpu.sync_copy(i2s.at[pl.ds(so, sz)], rows1.at[pl.ds(0, sz)])
        pltpu.sync_copy(rows1.at[pl.ds(0, sz)], y2.at[pl.ds(do, sz)])

    f0 = sid * 312
    fill_y2_lower(f0, 128)
    fill_y2_lower(f0 + 128, 128)
    fill_y2_lower(f0 + 256, 56)

    @pl.when(sid == 15)
    def _():
        fill_y2_lower(I - 8, 8)

    plsc.subcore_barrier()

    SS = 8

    def edge_stage(nblk, tbl, s_hbm, d_hbm, v_hbm):
        nss = nblk // SS
        hbase = sid * nblk

        def slab_descs(slot, ssidx):
            b = hbase + ssidx * SS
            return (
                pltpu.make_async_copy(s_hbm.at[cid].at[pl.ds(b, SS)],
                                      srcb.at[slot], lsem[slot]),
                pltpu.make_async_copy(d_hbm.at[pl.ds(b, SS)],
                                      dstb.at[slot], lsem[slot]),
                pltpu.make_async_copy(v_hbm.at[pl.ds(b, SS)],
                                      valb.at[slot], lsem[slot]),
            )

        def load_slab(slot, ssidx):
            for dsc in slab_descs(slot, ssidx):
                dsc.start()

        def wait_slab(slot, ssidx):
            for dsc in slab_descs(slot, ssidx):
                dsc.wait()

        def gdesc(s, slot, off):
            return pltpu.make_async_copy(tbl.at[srcb.at[slot].at[off]],
                                         rows[s], gsem[s])

        load_slab(0, 0)
        wait_slab(0, 0)
        load_slab(1, 1)
        gdesc(0, 0, 0).start()
        gdesc(1, 0, 1).start()

        def q_body(q, c):
            for sp in range(2):
                ss = 2 * q + sp

                def h_body(h, cc):
                    for s in range(2):
                        off = 2 * h + s

                        @pl.when(jnp.logical_and(off == SS - 2, ss + 1 < nss))
                        def _():
                            wait_slab(sp ^ 1, ss + 1)

                        gdesc(s, sp, off).wait()

                        def mul_body(m, ccc):
                            vv = valb[sp, off, pl.ds(m * 16, 16)]
                            for k in range(16):
                                v = vv[k]
                                kk = m * 16 + k
                                for j in range(8):
                                    sl = pl.ds(j * 16, 16)
                                    rows[s][kk, sl] = rows[s][kk, sl] * v
                            return ccc

                        lax.fori_loop(0, _BLK // 16, mul_body, 0)
                        pltpu.sync_copy(rows[s], acc.at[dstb.at[sp].at[off]],
                                        add=True)

                        @pl.when(off < SS - 2)
                        def _():
                            gdesc(s, sp, off + 2).start()

                        @pl.when(jnp.logical_and(off >= SS - 2,
                                                 ss * SS + off + 2 < nblk))
                        def _():
                            gdesc(s, sp ^ 1, off + 2 - SS).start()
                    return cc

                lax.fori_loop(0, SS // 2, h_body, 0)

                @pl.when(ss + 2 < nss)
                def _():
                    load_slab(sp, ss + 2)
            return c

        lax.fori_loop(0, nss // 2, q_body, 0)

    edge_stage(_PT_A, tab1, sA, dA, vA)
    plsc.subcore_barrier()

    def pub_y2(off, sz):
        pltpu.sync_copy(acc.at[pl.ds(off, sz)], rows0.at[pl.ds(0, sz)])
        pltpu.sync_copy(rows0.at[pl.ds(0, sz)], y2.at[pl.ds(cid * N + off, sz)])

    p0 = sid * 312
    pub_y2(p0, 128)
    pub_y2(p0 + 128, 128)
    pub_y2(p0 + 256, 56)

    @pl.when(sid == 15)
    def _():
        pub_y2(U - 8, 8)

    plsc.subcore_barrier()

    edge_stage(_PT_B, tab1, sB, dB, vB)
    edge_stage(_PT_H, y2, sH, dA, vA)
    plsc.subcore_barrier()

    def wout(off, sz):
        pltpu.sync_copy(acc.at[pl.ds(off, sz)], rows0.at[pl.ds(0, sz)])
        pltpu.sync_copy(rows0.at[pl.ds(0, sz)], out.at[pl.ds(cid * N + off, sz)])

    for i in range(4):
        wout(r0 + i * 128, 128)
    wout(r0 + 512, 112)

    @pl.when(sid == 15)
    def _():
        wout(N - 16, 16)


def _sc_spmm(tab1, i2s, sA, dA, vA, sB, dB, vB, sH):
    mesh = plsc.VectorSubcoreMesh(core_axis_name="c", subcore_axis_name="s")
    f = pl.kernel(
        _sc_body,
        mesh=mesh,
        out_type=(jax.ShapeDtypeStruct((2 * N, HD), jnp.float32),
                  jax.ShapeDtypeStruct((2 * N, HD), jnp.float32)),
        scratch_types=[
            pltpu.VMEM((2, 8, _BLK), jnp.int32),
            pltpu.VMEM((2, 8, _BLK), jnp.int32),
            pltpu.VMEM((2, 8, _BLK), jnp.float32),
            pltpu.VMEM((_BLK, HD), jnp.float32),
            pltpu.VMEM((_BLK, HD), jnp.float32),
            pltpu.VMEM_SHARED((N, HD), jnp.float32),
            pltpu.SemaphoreType.DMA,
            pltpu.SemaphoreType.DMA,
            pltpu.SemaphoreType.DMA,
            pltpu.SemaphoreType.DMA,
        ],
    )
    return f(tab1, i2s, sA, dA, vA, sB, dB, vB, sH)


def kernel(adj_indices, adj_values, image_adj_indices, image_adj_values,
           text_adj_indices, text_adj_values, u_embs, i_embs, image_embedding,
           text_embedding, image_W, image_b, text_W, text_b, modal_weight):
    weight = jax.nn.softmax(modal_weight, axis=0)
    w0, w1 = weight[0], weight[1]
    ws = w0 + w1

    lower = _modal_lower(weight, image_embedding, image_W, image_b,
                         text_W, text_b, text_embedding)

    x1 = jnp.concatenate([ws * u_embs, lower], axis=0)
    base = jnp.concatenate([u_embs, i_embs], axis=0)
    tab1 = jnp.concatenate([x1[:, :HD], base[:, :HD],
                            x1[:, HD:], base[:, HD:]], axis=0)
    i2 = ws * i_embs
    i2s = jnp.concatenate([i2[:, :HD], i2[:, HD:]], axis=0)

    a_dst, a_src = adj_indices[0], adj_indices[1]
    padA = _EA - NNZ
    srcA = jnp.pad(a_src, (0, padA))
    sA = jnp.stack([srcA, srcA + 2 * N]).reshape(2, _PT_A * _NT, _BLK)
    dA = jnp.pad(a_dst, (0, padA)).reshape(_PT_A * _NT, _BLK)
    vA = jnp.pad(adj_values, (0, padA)).reshape(_PT_A * _NT, _BLK)

    pq_src = jnp.concatenate([image_adj_indices[1], text_adj_indices[1]]) + N
    pq_dst = jnp.concatenate([image_adj_indices[0], text_adj_indices[0]])
    pq_val = jnp.concatenate([MODAL_ADJ_W * w0 * image_adj_values,
                              MODAL_ADJ_W * w1 * text_adj_values])
    padB = _EB - 2 * NNZ
    srcB = jnp.pad(pq_src, (0, padB))
    sB = jnp.stack([srcB, srcB + 2 * N]).reshape(2, _PT_B * _NT, _BLK)
    dB = jnp.pad(pq_dst, (0, padB)).reshape(_PT_B * _NT, _BLK)
    vB = jnp.pad(pq_val, (0, padB)).reshape(_PT_B * _NT, _BLK)

    sH = jnp.stack([srcA, srcA + N]).reshape(2, _PT_A * _NT, _BLK)

    o, _ = _sc_spmm(tab1, i2s, sA, dA, vA, sB, dB, vB, sH)
    return jnp.concatenate([o[:N], o[N:]], axis=1)


# speedup vs baseline: 3.6378x; 1.0010x over previous
import functools

import jax
import jax.numpy as jnp
from jax import lax
from jax.experimental import pallas as pl
from jax.experimental.pallas import tpu as pltpu
from jax.experimental.pallas import tpu_sc as plsc

U = 5000
I = 5000
N = 10000
D = 256
HD = 128
IMG_DIM = 4096
TXT_DIM = 768
NNZ = 160000
MODAL_ADJ_W = 0.2

_MB = 1000

_BLK = 80
_SS = 8
_PT_A = 128
_PT_B = 256
_PT_H = 128
_NT = 16
_EA = _PT_A * _NT * _BLK
_EB = _PT_B * _NT * _BLK
_RPT = 624


def _dense_body(w_ref, img_ref, iw_ref, ib_ref, txt_ref, tw_ref, tb_ref, out_ref):
    w0 = w_ref[0, 0]
    w1 = w_ref[0, 1]
    imf = jnp.dot(img_ref[...], iw_ref[...],
                  preferred_element_type=jnp.float32) + ib_ref[...]
    inorm = jnp.maximum(jnp.sqrt(jnp.sum(imf * imf, axis=1, keepdims=True)), 1e-12)
    txf = jnp.dot(txt_ref[...], tw_ref[...],
                  preferred_element_type=jnp.float32) + tb_ref[...]
    tnorm = jnp.maximum(jnp.sqrt(jnp.sum(txf * txf, axis=1, keepdims=True)), 1e-12)
    out_ref[...] = (w0 / inorm) * imf + (w1 / tnorm) * txf


def _modal_lower(weight, image_embedding, image_W, image_b, text_W, text_b,
                 text_embedding):
    return pl.pallas_call(
        _dense_body,
        grid=(I // _MB,),
        in_specs=[
            pl.BlockSpec(memory_space=pltpu.SMEM),
            pl.BlockSpec((_MB, IMG_DIM), lambda m: (m, 0)),
            pl.BlockSpec((IMG_DIM, D), lambda m: (0, 0)),
            pl.BlockSpec((1, D), lambda m: (0, 0)),
            pl.BlockSpec((_MB, TXT_DIM), lambda m: (m, 0)),
            pl.BlockSpec((TXT_DIM, D), lambda m: (0, 0)),
            pl.BlockSpec((1, D), lambda m: (0, 0)),
        ],
        out_specs=pl.BlockSpec((_MB, D), lambda m: (m, 0)),
        out_shape=jax.ShapeDtypeStruct((I, D), jnp.float32),
    )(weight.reshape(1, 2), image_embedding, image_W, image_b.reshape(1, D),
      text_embedding, text_W, text_b.reshape(1, D))


def _sc_body(tab1, i2s, sA, dA, vA, sB, dB, vB, sH,
             out, y2,
             srcb, dstb, valb, rows0, rows1, rows2, rows3, acc,
             g0, g1, g2, g3, c0, c1, c2, c3, l0, l1):
    cid = lax.axis_index("c")
    sid = lax.axis_index("s")
    rows = (rows0, rows1, rows2, rows3)
    gsem = (g0, g1, g2, g3)
    csem = (c0, c1, c2, c3)
    lsem = (l0, l1)

    z16 = jnp.zeros((16,), jnp.float32)

    def zrow(i, c):
        for j in range(8):
            rows0[i, pl.ds(j * 16, 16)] = z16
        return c

    lax.fori_loop(0, _BLK, zrow, 0)

    def zero_rows(off, sz):
        pltpu.sync_copy(rows0.at[pl.ds(0, sz)], acc.at[pl.ds(off, sz)])

    r0 = sid * _RPT
    for i in range(7):
        zero_rows(r0 + i * 80, 80)
    zero_rows(r0 + 560, 64)

    @pl.when(sid == 15)
    def _():
        zero_rows(N - 16, 16)

    def fill_y2_lower(off, sz):
        so = cid * I + off
        do = cid * N + U + off
        pltpu.sync_copy(i2s.at[pl.ds(so, sz)], rows1.at[pl.ds(0, sz)])
        pltpu.sync_copy(rows1.at[pl.ds(0, sz)], y2.at[pl.ds(do, sz)])

    f0 = sid * 312
    for i in range(3):
        fill_y2_lower(f0 + i * 80, 80)
    fill_y2_lower(f0 + 240, 72)

    @pl.when(sid == 15)
    def _():
        fill_y2_lower(I - 8, 8)

    plsc.subcore_barrier()

    def edge_stage(nblk, tbl, s_hbm, d_hbm, v_hbm):
        nss = nblk // _SS
        hbase = sid * nblk

        def slab_descs(slot, ssidx):
            b = hbase + ssidx * _SS
            return (
                pltpu.make_async_copy(s_hbm.at[cid].at[pl.ds(b, _SS)],
                                      srcb.at[slot], lsem[slot]),
                pltpu.make_async_copy(d_hbm.at[pl.ds(b, _SS)],
                                      dstb.at[slot], lsem[slot]),
                pltpu.make_async_copy(v_hbm.at[pl.ds(b, _SS)],
                                      valb.at[slot], lsem[slot]),
            )

        def load_slab(slot, ssidx):
            for dsc in slab_descs(slot, ssidx):
                dsc.start()

        def wait_slab(slot, ssidx):
            for dsc in slab_descs(slot, ssidx):
                dsc.wait()

        def gdesc(s, slot, off):
            return pltpu.make_async_copy(tbl.at[srcb.at[slot].at[off]],
                                         rows[s], gsem[s])

        def sdesc(s, slot, off):
            return pltpu.make_async_copy(rows[s], acc.at[dstb.at[slot].at[off]],
                                         csem[s])

        load_slab(0, 0)
        wait_slab(0, 0)
        load_slab(1, 1)
        gdesc(0, 0, 0).start()
        gdesc(1, 0, 1).start()

        def q_body(q, c):
            for sp in range(2):
                ss = 2 * q + sp

                def h_body(h, cc):
                    for u in range(4):
                        off = 4 * h + u
                        blk = ss * _SS + off
                        s = u
                        s2 = (u + 2) % 4

                        @pl.when(jnp.logical_and(off == _SS - 2, ss + 1 < nss))
                        def _():
                            wait_slab(sp ^ 1, ss + 1)

                        gdesc(s, sp, off).wait()

                        def mul_body(m, ccc):
                            vv = valb[sp, off, pl.ds(m * 16, 16)]
                            for k in range(16):
                                v = vv[k]
                                kk = m * 16 + k
                                for j in range(8):
                                    sl = pl.ds(j * 16, 16)
                                    rows[s][kk, sl] = rows[s][kk, sl] * v
                            return ccc

                        lax.fori_loop(0, _BLK // 16, mul_body, 0)
                        pltpu.async_copy(rows[s], acc.at[dstb.at[sp].at[off]],
                                         csem[s], add=True)

                        @pl.when(blk >= 2)
                        def _():
                            sdesc(s2, sp, off).wait()

                        @pl.when(jnp.logical_and(off < _SS - 2, blk + 2 < nblk))
                        def _():
                            gdesc(s2, sp, off + 2).start()

                        @pl.when(jnp.logical_and(off >= _SS - 2,
                                                 blk + 2 < nblk))
                        def _():
                            gdesc(s2, sp ^ 1, off + 2 - _SS).start()
                    return cc

                lax.fori_loop(0, _SS // 4, h_body, 0)

                @pl.when(ss + 2 < nss)
                def _():
                    load_slab(sp, ss + 2)
            return c

        lax.fori_loop(0, nss // 2, q_body, 0)
        for s2 in ((nblk - 2) % 4, (nblk - 1) % 4):
            sdesc(s2, 0, 0).wait()

    edge_stage(_PT_A, tab1, sA, dA, vA)
    plsc.subcore_barrier()

    def pub_y2(off, sz):
        pltpu.sync_copy(acc.at[pl.ds(off, sz)], rows0.at[pl.ds(0, sz)])
        pltpu.sync_copy(rows0.at[pl.ds(0, sz)], y2.at[pl.ds(cid * N + off, sz)])

    p0 = sid * 312
    for i in range(3):
        pub_y2(p0 + i * 80, 80)
    pub_y2(p0 + 240, 72)

    @pl.when(sid == 15)
    def _():
        pub_y2(U - 8, 8)

    plsc.subcore_barrier()

    edge_stage(_PT_B, tab1, sB, dB, vB)
    edge_stage(_PT_H, y2, sH, dA, vA)
    plsc.subcore_barrier()

    def wout(off, sz):
        pltpu.sync_copy(acc.at[pl.ds(off, sz)], rows0.at[pl.ds(0, sz)])
        pltpu.sync_copy(rows0.at[pl.ds(0, sz)], out.at[pl.ds(cid * N + off, sz)])

    for i in range(7):
        wout(r0 + i * 80, 80)
    wout(r0 + 560, 64)

    @pl.when(sid == 15)
    def _():
        wout(N - 16, 16)


def _sc_spmm(tab1, i2s, sA, dA, vA, sB, dB, vB, sH):
    mesh = plsc.VectorSubcoreMesh(core_axis_name="c", subcore_axis_name="s")
    f = pl.kernel(
        _sc_body,
        mesh=mesh,
        out_type=(jax.ShapeDtypeStruct((2 * N, HD), jnp.float32),
                  jax.ShapeDtypeStruct((2 * N, HD), jnp.float32)),
        scratch_types=[
            pltpu.VMEM((2, _SS, _BLK), jnp.int32),
            pltpu.VMEM((2, _SS, _BLK), jnp.int32),
            pltpu.VMEM((2, _SS, _BLK), jnp.float32),
            pltpu.VMEM((_BLK, HD), jnp.float32),
            pltpu.VMEM((_BLK, HD), jnp.float32),
            pltpu.VMEM((_BLK, HD), jnp.float32),
            pltpu.VMEM((_BLK, HD), jnp.float32),
            pltpu.VMEM_SHARED((N, HD), jnp.float32),
            pltpu.SemaphoreType.DMA,
            pltpu.SemaphoreType.DMA,
            pltpu.SemaphoreType.DMA,
            pltpu.SemaphoreType.DMA,
            pltpu.SemaphoreType.DMA,
            pltpu.SemaphoreType.DMA,
            pltpu.SemaphoreType.DMA,
            pltpu.SemaphoreType.DMA,
            pltpu.SemaphoreType.DMA,
            pltpu.SemaphoreType.DMA,
        ],
    )
    return f(tab1, i2s, sA, dA, vA, sB, dB, vB, sH)


def kernel(adj_indices, adj_values, image_adj_indices, image_adj_values,
           text_adj_indices, text_adj_values, u_embs, i_embs, image_embedding,
           text_embedding, image_W, image_b, text_W, text_b, modal_weight):
    weight = jax.nn.softmax(modal_weight, axis=0)
    w0, w1 = weight[0], weight[1]
    ws = w0 + w1

    lower = _modal_lower(weight, image_embedding, image_W, image_b,
                         text_W, text_b, text_embedding)

    x1 = jnp.concatenate([ws * u_embs, lower], axis=0)
    base = jnp.concatenate([u_embs, i_embs], axis=0)
    tab1 = jnp.concatenate([x1[:, :HD], base[:, :HD],
                            x1[:, HD:], base[:, HD:]], axis=0)
    i2 = ws * i_embs
    i2s = jnp.concatenate([i2[:, :HD], i2[:, HD:]], axis=0)

    a_dst, a_src = adj_indices[0], adj_indices[1]
    padA = _EA - NNZ
    srcA = jnp.pad(a_src, (0, padA))
    sA = jnp.stack([srcA, srcA + 2 * N]).reshape(2, _PT_A * _NT, _BLK)
    dA = jnp.pad(a_dst, (0, padA)).reshape(_PT_A * _NT, _BLK)
    vA = jnp.pad(adj_values, (0, padA)).reshape(_PT_A * _NT, _BLK)

    pq_src = jnp.concatenate([image_adj_indices[1], text_adj_indices[1]]) + N
    pq_dst = jnp.concatenate([image_adj_indices[0], text_adj_indices[0]])
    pq_val = jnp.concatenate([MODAL_ADJ_W * w0 * image_adj_values,
                              MODAL_ADJ_W * w1 * text_adj_values])
    padB = _EB - 2 * NNZ
    srcB = jnp.pad(pq_src, (0, padB))
    sB = jnp.stack([srcB, srcB + 2 * N]).reshape(2, _PT_B * _NT, _BLK)
    dB = jnp.pad(pq_dst, (0, padB)).reshape(_PT_B * _NT, _BLK)
    vB = jnp.pad(pq_val, (0, padB)).reshape(_PT_B * _NT, _BLK)

    sH = jnp.stack([srcA, srcA + N]).reshape(2, _PT_A * _NT, _BLK)

    o, _ = _sc_spmm(tab1, i2s, sA, dA, vA, sB, dB, vB, sH)
    return jnp.concatenate([o[:N], o[N:]], axis=1)
